# baseline (speedup 1.0000x reference)
import functools

import jax
import jax.numpy as jnp
from jax import lax
from jax.experimental import pallas as pl
from jax.experimental.pallas import tpu as pltpu
from jax.experimental.pallas import tpu_sc as plsc

_NUM_PATCHES = 4096
_EMBED_DIM = 1024
_BATCH = 4

_SC_B0 = 2
_SC_NB = 2
_TC_NB = _SC_B0

_NC = 2
_NS = 16
_NW = _NC * _NS
_LANES = 16

_ROWS_W = _NUM_PATCHES // _NW
_CH_ROWS = 8
_NCHUNK = _ROWS_W // _CH_ROWS
_VECS = _CH_ROWS * _EMBED_DIM // _LANES
_VEC_ROW = _EMBED_DIM // _LANES


def _add_chunk(ob, pb, pv):
    @plsc.parallel_loop(0, _VECS, unroll=8)
    def _(i):
        r = i >> 6
        sl = pl.ds((i & (_VEC_ROW - 1)) * _LANES, _LANES)
        ob[r, sl] = pb[r, sl] + pv[r, sl]


def _sc_body(patch_hbm, pos_hbm, out_hbm, *scratch):
    pbuf = scratch[0:_SC_NB]
    obuf = scratch[_SC_NB:2 * _SC_NB]
    pos_v = scratch[2 * _SC_NB:2 * _SC_NB + 2]
    sem_in = scratch[2 * _SC_NB + 2:3 * _SC_NB + 2]
    sem_out = scratch[3 * _SC_NB + 2:4 * _SC_NB + 2]
    sem_pos = scratch[4 * _SC_NB + 2:4 * _SC_NB + 4]

    c_ax = lax.axis_index("c")
    s_ax = lax.axis_index("s")
    wid = s_ax * _NC + c_ax
    base = wid * _ROWS_W

    def issue_pos(c, par):
        pltpu.async_copy(
            pos_hbm.at[pl.ds(base + c * _CH_ROWS, _CH_ROWS), :], pos_v[par],
            sem_pos[par])

    def wait_pos(par):
        pltpu.make_async_copy(
            pos_hbm.at[pl.ds(0, _CH_ROWS), :], pos_v[par],
            sem_pos[par]).wait()

    def issue_in(c, b):
        row = base + c * _CH_ROWS
        pltpu.async_copy(
            patch_hbm.at[_SC_B0 + b, pl.ds(row, _CH_ROWS), :], pbuf[b],
            sem_in[b])

    def wait_in(b):
        pltpu.make_async_copy(
            patch_hbm.at[0, pl.ds(0, _CH_ROWS), :], pbuf[b],
            sem_in[b]).wait()

    def issue_out(c, b):
        row = base + c * _CH_ROWS
        pltpu.async_copy(obuf[b], out_hbm.at[b, pl.ds(row, _CH_ROWS), :],
                         sem_out[b])

    def wait_out(b):
        pltpu.make_async_copy(
            obuf[b], out_hbm.at[0, pl.ds(0, _CH_ROWS), :],
            sem_out[b]).wait()

    for b in range(_SC_NB):
        issue_in(0, b)
    issue_pos(0, 0)
    issue_pos(1, 1)

    def pair(h, _):
        c0 = 2 * h
        c1 = c0 + 1

        wait_pos(0)
        for b in range(_SC_NB):
            wait_in(b)
            pl.when(h > 0)(lambda b=b: wait_out(b))
            _add_chunk(obuf[b], pbuf[b], pos_v[0])
            issue_in(c1, b)
            issue_out(c0, b)
        pl.when(h < _NCHUNK // 2 - 1)(lambda: issue_pos(c0 + 2, 0))

        wait_pos(1)
        for b in range(_SC_NB):
            wait_in(b)
            wait_out(b)
            _add_chunk(obuf[b], pbuf[b], pos_v[1])
            pl.when(h < _NCHUNK // 2 - 1)(lambda b=b: issue_in(c1 + 1, b))
            issue_out(c1, b)
        pl.when(h < _NCHUNK // 2 - 1)(lambda: issue_pos(c1 + 2, 1))
        return None

    lax.fori_loop(0, _NCHUNK // 2, pair, None)

    for b in range(_SC_NB):
        wait_out(b)


def _sc_half(patch, pos_table):
    mesh = plsc.VectorSubcoreMesh(core_axis_name="c", subcore_axis_name="s")
    return pl.kernel(
        _sc_body,
        out_type=jax.ShapeDtypeStruct((_SC_NB, _NUM_PATCHES, _EMBED_DIM),
                                      jnp.float32),
        mesh=mesh,
        scratch_types=(
            [pltpu.VMEM((_CH_ROWS, _EMBED_DIM), jnp.float32)
             for _ in range(2 * _SC_NB + 2)]
            + [pltpu.SemaphoreType.DMA for _ in range(2 * _SC_NB + 2)]
        ),
    )(patch, pos_table)


_TC_ROWS = 512


def _tc_body(patch_ref, pos_ref, out_ref):
    out_ref[0] = patch_ref[0] + pos_ref[...]


def _tc_half(patch, pos_table):
    grid = (_TC_NB, _NUM_PATCHES // _TC_ROWS)
    return pl.pallas_call(
        _tc_body,
        grid=grid,
        in_specs=[
            pl.BlockSpec((1, _TC_ROWS, _EMBED_DIM), lambda b, i: (b, i, 0)),
            pl.BlockSpec((_TC_ROWS, _EMBED_DIM), lambda b, i: (i, 0)),
        ],
        out_specs=pl.BlockSpec((1, _TC_ROWS, _EMBED_DIM),
                               lambda b, i: (b, i, 0)),
        out_shape=jax.ShapeDtypeStruct((_TC_NB, _NUM_PATCHES, _EMBED_DIM),
                                       jnp.float32),
        compiler_params=pltpu.CompilerParams(
            dimension_semantics=("arbitrary", "arbitrary")),
    )(patch, pos_table)


@jax.jit
def kernel(patch, pos_table):
    sc_out = _sc_half(patch, pos_table)
    tc_out = _tc_half(patch, pos_table)
    return lax.concatenate([tc_out, sc_out], 0)


# speedup vs baseline: 1.5871x; 1.5871x over previous
import jax
import jax.numpy as jnp
from jax import lax
from jax.experimental import pallas as pl
from jax.experimental.pallas import tpu as pltpu
from jax.experimental.pallas import tpu_sc as plsc

_NUM_PATCHES = 4096
_EMBED_DIM = 1024
_BATCH = 4

_NC = 2
_NS = 16
_NW = _NC * _NS
_LANES = 16

_ROWS_W = _NUM_PATCHES // _NW
_CH_ROWS = 16
_H_ROWS = _CH_ROWS // 2
_NCHUNK = _ROWS_W // _CH_ROWS
_HVECS = _H_ROWS * _EMBED_DIM // _LANES
_VEC_ROW = _EMBED_DIM // _LANES


def _add_half(ob, pb, pv, half):
    r0 = half * _H_ROWS

    @plsc.parallel_loop(0, _HVECS, unroll=8)
    def _(i):
        r = i >> 6
        sl = pl.ds((i & (_VEC_ROW - 1)) * _LANES, _LANES)
        ob[r, sl] = pb[r0 + r, sl] + pv[r0 + r, sl]


def _body(patch_hbm, pos_hbm, out_hbm, *scratch):
    pbuf = scratch[0:4]
    obuf = scratch[4:6]
    pos_v = scratch[6:8]
    sem_in = scratch[8:12]
    sem_out = scratch[12:14]
    sem_pos = scratch[14:16]

    c_ax = lax.axis_index("c")
    s_ax = lax.axis_index("s")
    wid = s_ax * _NC + c_ax
    base = wid * _ROWS_W

    def issue_pos(c, par):
        pltpu.async_copy(
            pos_hbm.at[pl.ds(base + c * _CH_ROWS, _CH_ROWS), :], pos_v[par],
            sem_pos[par])

    def wait_pos(par):
        pltpu.make_async_copy(
            pos_hbm.at[pl.ds(0, _CH_ROWS), :], pos_v[par],
            sem_pos[par]).wait()

    def issue_in(c, b):
        row = base + c * _CH_ROWS
        pltpu.async_copy(patch_hbm.at[b, pl.ds(row, _CH_ROWS), :], pbuf[b],
                         sem_in[b])

    def wait_in(b):
        pltpu.make_async_copy(
            patch_hbm.at[0, pl.ds(0, _CH_ROWS), :], pbuf[b],
            sem_in[b]).wait()

    def issue_out(c, b, half):
        row = base + c * _CH_ROWS + half * _H_ROWS
        pltpu.async_copy(obuf[half], out_hbm.at[b, pl.ds(row, _H_ROWS), :],
                         sem_out[half])

    def wait_out(half):
        pltpu.make_async_copy(
            obuf[half], out_hbm.at[0, pl.ds(0, _H_ROWS), :],
            sem_out[half]).wait()

    for b in range(_BATCH):
        issue_in(0, b)
    issue_pos(0, 0)
    issue_pos(1, 1)

    last = _NCHUNK // 2 - 1

    def chunk_sched(h, c, par, in_guarded):
        wait_pos(par)
        for b in range(_BATCH):
            wait_in(b)
            for half in range(2):
                if par == 0 and b == 0:
                    pl.when(h > 0)(lambda half=half: wait_out(half))
                else:
                    wait_out(half)
                _add_half(obuf[half], pbuf[b], pos_v[par], half)
                issue_out(c, b, half)
            if in_guarded:
                pl.when(h < last)(lambda b=b: issue_in(c + 1, b))
            else:
                issue_in(c + 1, b)
        pl.when(h < last)(lambda: issue_pos(c + 2, par))

    def pair(h, _):
        c0 = 2 * h
        chunk_sched(h, c0, 0, in_guarded=False)
        chunk_sched(h, c0 + 1, 1, in_guarded=True)
        return None

    lax.fori_loop(0, _NCHUNK // 2, pair, None)

    for half in range(2):
        wait_out(half)


@jax.jit
def kernel(patch, pos_table):
    mesh = plsc.VectorSubcoreMesh(core_axis_name="c", subcore_axis_name="s")
    return pl.kernel(
        _body,
        out_type=jax.ShapeDtypeStruct((_BATCH, _NUM_PATCHES, _EMBED_DIM),
                                      jnp.float32),
        mesh=mesh,
        scratch_types=(
            [pltpu.VMEM((_CH_ROWS, _EMBED_DIM), jnp.float32)
             for _ in range(4)]
            + [pltpu.VMEM((_H_ROWS, _EMBED_DIM), jnp.float32)
               for _ in range(2)]
            + [pltpu.VMEM((_CH_ROWS, _EMBED_DIM), jnp.float32)
               for _ in range(2)]
            + [pltpu.SemaphoreType.DMA for _ in range(8)]
        ),
    )(patch, pos_table)


# speedup vs baseline: 1.6329x; 1.0289x over previous
import jax
import jax.numpy as jnp
from jax import lax
from jax.experimental import pallas as pl
from jax.experimental.pallas import tpu as pltpu
from jax.experimental.pallas import tpu_sc as plsc

_NUM_PATCHES = 4096
_EMBED_DIM = 1024
_BATCH = 4

_NC = 2
_NS = 16
_NW = _NC * _NS
_LANES = 16

_ROWS_W = _NUM_PATCHES // _NW
_CH_ROWS = 8
_NCHUNK = _ROWS_W // _CH_ROWS
_VECS = _CH_ROWS * _EMBED_DIM // _LANES
_VEC_ROW = _EMBED_DIM // _LANES


def _add_chunk(ob, pb, pv):
    @plsc.parallel_loop(0, _VECS, unroll=8)
    def _(i):
        r = i >> 6
        sl = pl.ds((i & (_VEC_ROW - 1)) * _LANES, _LANES)
        ob[r, sl] = pb[r, sl] + pv[r, sl]


def _body(patch_hbm, pos_hbm, out_hbm, *scratch):
    pbuf = scratch[0:4]
    obuf = scratch[4:8]
    pos_v = scratch[8:10]
    sem_in = scratch[10:14]
    sem_out = scratch[14:18]
    sem_pos = scratch[18:20]

    c_ax = lax.axis_index("c")
    s_ax = lax.axis_index("s")
    wid = s_ax * _NC + c_ax
    base = wid * _ROWS_W

    def issue_pos(c, par):
        pltpu.async_copy(
            pos_hbm.at[pl.ds(base + c * _CH_ROWS, _CH_ROWS), :], pos_v[par],
            sem_pos[par])

    def wait_pos(par):
        pltpu.make_async_copy(
            pos_hbm.at[pl.ds(0, _CH_ROWS), :], pos_v[par],
            sem_pos[par]).wait()

    def issue_in(c, b):
        row = base + c * _CH_ROWS
        pltpu.async_copy(patch_hbm.at[b, pl.ds(row, _CH_ROWS), :], pbuf[b],
                         sem_in[b])

    def wait_in(b):
        pltpu.make_async_copy(
            patch_hbm.at[0, pl.ds(0, _CH_ROWS), :], pbuf[b],
            sem_in[b]).wait()

    def issue_out(c, b):
        row = base + c * _CH_ROWS
        pltpu.async_copy(obuf[b], out_hbm.at[b, pl.ds(row, _CH_ROWS), :],
                         sem_out[b])

    def wait_out(b):
        pltpu.make_async_copy(
            obuf[b], out_hbm.at[0, pl.ds(0, _CH_ROWS), :],
            sem_out[b]).wait()

    for b in range(_BATCH):
        issue_in(0, b)
    issue_pos(0, 0)
    issue_pos(1, 1)

    def pair(h, _):
        c0 = 2 * h
        c1 = c0 + 1

        wait_pos(0)
        for b in range(_BATCH):
            wait_in(b)
            pl.when(h > 0)(lambda b=b: wait_out(b))
            _add_chunk(obuf[b], pbuf[b], pos_v[0])
            issue_in(c1, b)
            issue_out(c0, b)
        pl.when(h < _NCHUNK // 2 - 1)(lambda: issue_pos(c0 + 2, 0))

        wait_pos(1)
        for b in range(_BATCH):
            wait_in(b)
            wait_out(b)
            _add_chunk(obuf[b], pbuf[b], pos_v[1])
            pl.when(h < _NCHUNK // 2 - 1)(lambda b=b: issue_in(c1 + 1, b))
            issue_out(c1, b)
        pl.when(h < _NCHUNK // 2 - 1)(lambda: issue_pos(c1 + 2, 1))
        return None

    lax.fori_loop(0, _NCHUNK // 2, pair, None)

    for b in range(_BATCH):
        wait_out(b)


@jax.jit
def kernel(patch, pos_table):
    mesh = plsc.VectorSubcoreMesh(core_axis_name="c", subcore_axis_name="s")
    return pl.kernel(
        _body,
        out_type=jax.ShapeDtypeStruct((_BATCH, _NUM_PATCHES, _EMBED_DIM),
                                      jnp.float32),
        mesh=mesh,
        scratch_types=(
            [pltpu.VMEM((_CH_ROWS, _EMBED_DIM), jnp.float32)
             for _ in range(10)]
            + [pltpu.SemaphoreType.DMA for _ in range(10)]
        ),
    )(patch, pos_table)
